# baseline (speedup 1.0000x reference)
import functools

import jax
import jax.numpy as jnp
from jax import lax
from jax.experimental import pallas as pl
from jax.experimental.pallas import tpu as pltpu
from jax.experimental.pallas import tpu_sc as plsc

NC = 2
NS = 16
NW = NC * NS
LANES = 16
CH = 4096


def _grid_consts(in_width):
    out_w = in_width * 4
    out_h = in_width * 2
    YY, XX = jnp.meshgrid(jnp.arange(out_h, dtype=jnp.float32),
                          jnp.arange(out_w, dtype=jnp.float32), indexing='ij')
    theta = (XX / out_w - 0.5) * 2.0 * jnp.pi
    phi = (YY / out_h - 0.5) * jnp.pi
    big = 1e30
    theta = jnp.clip(jnp.nan_to_num(theta, posinf=big, neginf=-big), -big, big)
    phi = jnp.clip(jnp.nan_to_num(phi, posinf=big, neginf=-big), -big, big)
    _x = jnp.cos(phi) * jnp.cos(theta)
    _y = jnp.cos(phi) * jnp.sin(theta)
    _z = jnp.sin(phi)
    ax, ay, az = jnp.abs(_x), jnp.abs(_y), jnp.abs(_z)
    face_map = jnp.where((ax >= ay) & (ax >= az),
                         jnp.where(_x >= 0, 0, 1),
                         jnp.where(ay >= az,
                                   jnp.where(_y >= 0, 2, 3),
                                   jnp.where(_z >= 0, 4, 5))).astype(jnp.int32)
    eps = 1e-9
    us = jnp.stack([-_y / (ax + eps), _y / (ax + eps), _x / (ay + eps),
                    -_x / (ay + eps), _x / (az + eps), -_x / (az + eps)], axis=0)
    vs = jnp.stack([-_z / (ax + eps), -_z / (ax + eps), -_z / (ay + eps),
                    -_z / (ay + eps), _y / (az + eps), _y / (az + eps)], axis=0)
    u = jnp.take_along_axis(us, face_map[None], axis=0)[0]
    v = jnp.take_along_axis(vs, face_map[None], axis=0)[0]
    x_o = (u + 1.0) * 0.5 * (in_width - 1)
    y_o = (v + 1.0) * 0.5 * (in_width - 1)
    out_coord = jnp.stack([x_o, y_o], axis=2)
    mx = jnp.max(out_coord)
    gridf = (out_coord - mx / 2.0) / (mx / 2.0)
    W = in_width
    gx = (gridf[..., 0] + 1.0) * 0.5 * (W - 1)
    gy = (gridf[..., 1] + 1.0) * 0.5 * (W - 1)
    x0 = jnp.floor(gx).astype(jnp.int32)
    y0 = jnp.floor(gy).astype(jnp.int32)
    x0c = jnp.minimum(x0, W - 2)
    y0c = jnp.minimum(y0, W - 2)
    wx = gx - x0c.astype(jnp.float32)
    wy = gy - y0c.astype(jnp.float32)
    idx = face_map * (W * W) + y0c * W + x0c
    qx = jnp.round(wx * 65535.0).astype(jnp.int32)
    qy = jnp.round(wy * 65535.0).astype(jnp.int32)
    wpk = (qx << 16) | qy
    return idx.ravel(), wpk.ravel()


def _sc_resample(inp_flat, iw, B, F, C, H, W, P):
    planes = B * C
    ppt = planes // NW
    nchunk = P // CH
    face = H * W
    tbl = F * face

    mesh = plsc.VectorSubcoreMesh(core_axis_name="c", subcore_axis_name="s")

    @functools.partial(
        pl.kernel,
        mesh=mesh,
        out_type=jax.ShapeDtypeStruct((planes * P,), jnp.float32),
        compiler_params=pltpu.CompilerParams(needs_layout_passes=False),
        scratch_types=[
            pltpu.VMEM((tbl,), jnp.float32),
            pltpu.VMEM((2, 2 * CH + LANES), jnp.int32),
            pltpu.VMEM((2, CH), jnp.float32),
            pltpu.SemaphoreType.DMA,
            pltpu.SemaphoreType.DMA,
            pltpu.SemaphoreType.DMA,
            pltpu.SemaphoreType.DMA,
            pltpu.SemaphoreType.DMA,
        ],
    )
    def run(inp_hbm, iw_hbm, out_hbm, table_v, iw_v, out_v,
            in_sem0, in_sem1, out_sem0, out_sem1, tbl_sem):
        in_sems = (in_sem0, in_sem1)
        out_sems = (out_sem0, out_sem1)
        wid = lax.axis_index("s") * NC + lax.axis_index("c")

        def start_in(g, hb):
            pltpu.async_copy(iw_hbm.at[pl.ds(g * (2 * CH), 2 * CH)],
                             iw_v.at[hb, pl.ds(0, 2 * CH)], in_sems[hb])

        def wait_in(hb):
            pltpu.make_async_copy(iw_hbm.at[pl.ds(0, 2 * CH)],
                                  iw_v.at[hb, pl.ds(0, 2 * CH)],
                                  in_sems[hb]).wait()

        def wait_out(hb):
            pltpu.make_async_copy(out_v.at[hb], out_hbm.at[pl.ds(0, CH)],
                                  out_sems[hb]).wait()

        for pi in range(ppt):
            plane = wid * ppt + pi
            b = plane // C
            c = plane - b * C
            for f in range(F):
                off = ((b * F + f) * C + c) * face
                pltpu.async_copy(inp_hbm.at[pl.ds(off, face)],
                                 table_v.at[pl.ds(f * face, face)], tbl_sem)
            start_in(0, 0)
            for f in range(F):
                pltpu.make_async_copy(inp_hbm.at[pl.ds(0, face)],
                                      table_v.at[pl.ds(0, face)],
                                      tbl_sem).wait()

            def chunk_pair(g2, _, plane=plane, pi=pi):
                for hb in range(2):
                    g = g2 * 2 + hb

                    @pl.when(g + 1 < nchunk)
                    def _():
                        start_in(g + 1, 1 - hb)

                    wait_in(hb)

                    @pl.when((g >= 2) | (pi > 0))
                    def _():
                        wait_out(hb)

                    first = (iw_v[hb, pl.ds(0, LANES)],
                             iw_v[hb, pl.ds(CH, LANES)])

                    @plsc.parallel_loop(0, CH, step=LANES, unroll=8,
                                        carry=first)
                    def _(s, carry, hb=hb):
                        base, pk = carry
                        nbase = iw_v[hb, pl.ds(s + LANES, LANES)]
                        npk = iw_v[hb, pl.ds(CH + s + LANES, LANES)]
                        scale = jnp.float32(1.0 / 65535.0)
                        wx = lax.shift_right_logical(pk, 16).astype(jnp.float32) * scale
                        wy = (pk & jnp.int32(0xFFFF)).astype(jnp.float32) * scale
                        v00 = plsc.load_gather(table_v, [base])
                        v01 = plsc.load_gather(table_v, [base + 1])
                        v10 = plsc.load_gather(table_v, [base + W])
                        v11 = plsc.load_gather(table_v, [base + (W + 1)])
                        top = v00 + wx * (v01 - v00)
                        bot = v10 + wx * (v11 - v10)
                        out_v[hb, pl.ds(s, LANES)] = top + wy * (bot - top)
                        return (nbase, npk)

                    pltpu.async_copy(out_v.at[hb],
                                     out_hbm.at[pl.ds(plane * P + g * CH, CH)],
                                     out_sems[hb])
                return 0

            lax.fori_loop(0, nchunk // 2, chunk_pair, 0)
        wait_out(0)
        wait_out(1)

    return run(inp_flat, iw)


def kernel(input_data):
    B, F, C, H, W = input_data.shape
    P = (2 * W) * (4 * W)
    idx, wpk = _grid_consts(W)
    nchunk = P // CH
    iw = jnp.concatenate([idx.reshape(nchunk, CH), wpk.reshape(nchunk, CH)],
                         axis=1).reshape(-1)
    out_flat = _sc_resample(input_data.reshape(-1), iw, B, F, C, H, W, P)
    return out_flat.reshape(B, C, 2 * W, 4 * W)


# speedup vs baseline: 1.1466x; 1.1466x over previous
import functools

import jax
import jax.numpy as jnp
from jax import lax
from jax.experimental import pallas as pl
from jax.experimental.pallas import tpu as pltpu
from jax.experimental.pallas import tpu_sc as plsc

NC = 2
NS = 16
NW = NC * NS
LANES = 16
CH = 2048


def _grid_consts(in_width):
    out_w = in_width * 4
    out_h = in_width * 2
    YY, XX = jnp.meshgrid(jnp.arange(out_h, dtype=jnp.float32),
                          jnp.arange(out_w, dtype=jnp.float32), indexing='ij')
    theta = (XX / out_w - 0.5) * 2.0 * jnp.pi
    phi = (YY / out_h - 0.5) * jnp.pi
    big = 1e30
    theta = jnp.clip(jnp.nan_to_num(theta, posinf=big, neginf=-big), -big, big)
    phi = jnp.clip(jnp.nan_to_num(phi, posinf=big, neginf=-big), -big, big)
    _x = jnp.cos(phi) * jnp.cos(theta)
    _y = jnp.cos(phi) * jnp.sin(theta)
    _z = jnp.sin(phi)
    ax, ay, az = jnp.abs(_x), jnp.abs(_y), jnp.abs(_z)
    face_map = jnp.where((ax >= ay) & (ax >= az),
                         jnp.where(_x >= 0, 0, 1),
                         jnp.where(ay >= az,
                                   jnp.where(_y >= 0, 2, 3),
                                   jnp.where(_z >= 0, 4, 5))).astype(jnp.int32)
    eps = 1e-9
    us = jnp.stack([-_y / (ax + eps), _y / (ax + eps), _x / (ay + eps),
                    -_x / (ay + eps), _x / (az + eps), -_x / (az + eps)], axis=0)
    vs = jnp.stack([-_z / (ax + eps), -_z / (ax + eps), -_z / (ay + eps),
                    -_z / (ay + eps), _y / (az + eps), _y / (az + eps)], axis=0)
    u = jnp.take_along_axis(us, face_map[None], axis=0)[0]
    v = jnp.take_along_axis(vs, face_map[None], axis=0)[0]
    x_o = (u + 1.0) * 0.5 * (in_width - 1)
    y_o = (v + 1.0) * 0.5 * (in_width - 1)
    out_coord = jnp.stack([x_o, y_o], axis=2)
    mx = jnp.max(out_coord)
    gridf = (out_coord - mx / 2.0) / (mx / 2.0)
    W = in_width
    gx = (gridf[..., 0] + 1.0) * 0.5 * (W - 1)
    gy = (gridf[..., 1] + 1.0) * 0.5 * (W - 1)
    x0 = jnp.floor(gx).astype(jnp.int32)
    y0 = jnp.floor(gy).astype(jnp.int32)
    x0c = jnp.minimum(x0, W - 2)
    y0c = jnp.minimum(y0, W - 2)
    wx = gx - x0c.astype(jnp.float32)
    wy = gy - y0c.astype(jnp.float32)
    idx = face_map * (W * W) + y0c * W + x0c
    qx = jnp.round(wx * 65535.0).astype(jnp.int32)
    qy = jnp.round(wy * 65535.0).astype(jnp.int32)
    wpk = (qx << 16) | qy
    return idx.ravel(), wpk.ravel()


def _sc_resample(inp_flat, idx, wpk, B, F, C, H, W, P):
    planes = B * C
    ppt = planes // NW
    nchunk = P // CH
    face = H * W
    tbl = F * face

    mesh = plsc.VectorSubcoreMesh(core_axis_name="c", subcore_axis_name="s")

    @functools.partial(
        pl.kernel,
        mesh=mesh,
        out_type=jax.ShapeDtypeStruct((planes * P,), jnp.float32),
        compiler_params=pltpu.CompilerParams(needs_layout_passes=False),
        scratch_types=[
            pltpu.VMEM((tbl,), jnp.float32),
            pltpu.VMEM((2, CH + LANES), jnp.int32),
            pltpu.VMEM((2, CH + LANES), jnp.int32),
            pltpu.VMEM((2, CH), jnp.float32),
            pltpu.VMEM_SHARED((P,), jnp.int32),
            pltpu.VMEM_SHARED((P,), jnp.int32),
            pltpu.SemaphoreType.DMA,
            pltpu.SemaphoreType.DMA,
            pltpu.SemaphoreType.DMA,
            pltpu.SemaphoreType.DMA,
            pltpu.SemaphoreType.DMA,
        ],
    )
    def run(inp_hbm, idx_hbm, wpk_hbm, out_hbm, table_v, idx_v, w_v, out_v,
            idx_s, wpk_s, in_sem0, in_sem1, out_sem0, out_sem1, tbl_sem):
        in_sems = (in_sem0, in_sem1)
        out_sems = (out_sem0, out_sem1)
        wid = lax.axis_index("s") * NC + lax.axis_index("c")
        sid = lax.axis_index("s")

        @pl.when(sid == 0)
        def _():
            pltpu.async_copy(idx_hbm, idx_s, tbl_sem)
            pltpu.async_copy(wpk_hbm, wpk_s, tbl_sem)
            pltpu.make_async_copy(idx_hbm, idx_s, tbl_sem).wait()
            pltpu.make_async_copy(wpk_hbm, wpk_s, tbl_sem).wait()

        plsc.subcore_barrier()

        def start_in(g, hb):
            pbase = g * CH
            pltpu.async_copy(idx_s.at[pl.ds(pbase, CH)],
                             idx_v.at[hb, pl.ds(0, CH)], in_sems[hb])
            pltpu.async_copy(wpk_s.at[pl.ds(pbase, CH)],
                             w_v.at[hb, pl.ds(0, CH)], in_sems[hb])

        def wait_in(hb):
            pltpu.make_async_copy(idx_s.at[pl.ds(0, CH)],
                                  idx_v.at[hb, pl.ds(0, CH)],
                                  in_sems[hb]).wait()
            pltpu.make_async_copy(wpk_s.at[pl.ds(0, CH)],
                                  w_v.at[hb, pl.ds(0, CH)],
                                  in_sems[hb]).wait()

        def wait_out(hb):
            pltpu.make_async_copy(out_v.at[hb], out_hbm.at[pl.ds(0, CH)],
                                  out_sems[hb]).wait()

        for pi in range(ppt):
            plane = wid * ppt + pi
            b = plane // C
            c = plane - b * C
            for f in range(F):
                off = ((b * F + f) * C + c) * face
                pltpu.async_copy(inp_hbm.at[pl.ds(off, face)],
                                 table_v.at[pl.ds(f * face, face)], tbl_sem)
            start_in(0, 0)
            for f in range(F):
                pltpu.make_async_copy(inp_hbm.at[pl.ds(0, face)],
                                      table_v.at[pl.ds(0, face)],
                                      tbl_sem).wait()

            def chunk_pair(g2, _, plane=plane, pi=pi):
                for hb in range(2):
                    g = g2 * 2 + hb

                    @pl.when(g + 1 < nchunk)
                    def _():
                        start_in(g + 1, 1 - hb)

                    wait_in(hb)

                    @pl.when((g >= 2) | (pi > 0))
                    def _():
                        wait_out(hb)

                    first = (idx_v[hb, pl.ds(0, LANES)],
                             w_v[hb, pl.ds(0, LANES)])

                    @plsc.parallel_loop(0, CH, step=LANES, unroll=8,
                                        carry=first)
                    def _(s, carry, hb=hb):
                        base, pk = carry
                        nbase = idx_v[hb, pl.ds(s + LANES, LANES)]
                        npk = w_v[hb, pl.ds(s + LANES, LANES)]
                        scale = jnp.float32(1.0 / 65535.0)
                        wx = lax.shift_right_logical(pk, 16).astype(jnp.float32) * scale
                        wy = (pk & jnp.int32(0xFFFF)).astype(jnp.float32) * scale
                        v00 = plsc.load_gather(table_v, [base])
                        v01 = plsc.load_gather(table_v, [base + 1])
                        v10 = plsc.load_gather(table_v, [base + W])
                        v11 = plsc.load_gather(table_v, [base + (W + 1)])
                        top = v00 + wx * (v01 - v00)
                        bot = v10 + wx * (v11 - v10)
                        out_v[hb, pl.ds(s, LANES)] = top + wy * (bot - top)
                        return (nbase, npk)

                    pltpu.async_copy(out_v.at[hb],
                                     out_hbm.at[pl.ds(plane * P + g * CH, CH)],
                                     out_sems[hb])
                return 0

            lax.fori_loop(0, nchunk // 2, chunk_pair, 0)
        wait_out(0)
        wait_out(1)

    return run(inp_flat, idx, wpk)


def kernel(input_data):
    B, F, C, H, W = input_data.shape
    P = (2 * W) * (4 * W)
    idx, wpk = _grid_consts(W)
    out_flat = _sc_resample(input_data.reshape(-1), idx, wpk, B, F, C, H, W, P)
    return out_flat.reshape(B, C, 2 * W, 4 * W)
